# baseline (speedup 1.0000x reference)
import functools

import jax
import jax.numpy as jnp
from jax import lax
from jax.experimental import pallas as pl
from jax.experimental.pallas import tpu as pltpu
from jax.experimental.pallas import tpu_sc as plsc

N = 10000
D = 128
E = 320000
NC = 2
NS = 16
NW = NC * NS
EPW = E // NW
K = 125
C = EPW // K
IB = 8
NB = C // IB
NROWS = 10240
RPT = NROWS // NS
DPT = NROWS // NS
SR = N // NS
ZR = 64

_mesh = plsc.VectorSubcoreMesh(core_axis_name="c", subcore_axis_name="s")
_sc_params = pltpu.CompilerParams(use_tc_tiling_on_sc=False)


@functools.partial(
    pl.kernel,
    out_type=jax.ShapeDtypeStruct((NC * NROWS,), jnp.float32),
    mesh=_mesh,
    compiler_params=_sc_params,
    scratch_types=dict(
        deg=pltpu.VMEM_SHARED((NROWS,), jnp.float32),
        dst_v=pltpu.VMEM((C, K), jnp.int32),
        ones_v=pltpu.VMEM((K,), jnp.float32),
        zbuf=pltpu.VMEM((DPT,), jnp.float32),
    ),
)
def _deg_kernel(ei_hbm, out_hbm, *, deg, dst_v, ones_v, zbuf):
    cid = lax.axis_index("c")
    sid = lax.axis_index("s")
    wid = sid * NC + cid

    def zbody(i, _):
        zbuf[pl.ds(i * 16, 16)] = jnp.zeros((16,), jnp.float32)
        return 0

    lax.fori_loop(0, DPT // 16, zbody, 0)
    pltpu.sync_copy(zbuf, deg.at[pl.ds(sid * DPT, DPT)])

    ones_offs = list(range(0, K - 15, 16))
    if ones_offs[-1] + 16 < K:
        ones_offs.append(K - 16)
    for off in ones_offs:
        ones_v[pl.ds(off, 16)] = jnp.ones((16,), jnp.float32)

    pltpu.sync_copy(ei_hbm.at[1, wid], dst_v)
    plsc.subcore_barrier()

    def chunk(j, _):
        pltpu.sync_copy(ones_v, deg.at[dst_v.at[j]], add=True)
        return 0

    lax.fori_loop(0, C, chunk, 0)
    plsc.subcore_barrier()

    pltpu.sync_copy(deg.at[pl.ds(sid * DPT, DPT)],
                    out_hbm.at[pl.ds(cid * NROWS + sid * DPT, DPT)])


@functools.partial(
    pl.kernel,
    out_type=jax.ShapeDtypeStruct((NC, NROWS, D), jnp.int16),
    mesh=_mesh,
    compiler_params=_sc_params,
    scratch_types=dict(
        acc=pltpu.VMEM_SHARED((NROWS, D), jnp.int16),
        src_v=pltpu.VMEM((2, IB, K), jnp.int32),
        dst_v=pltpu.VMEM((2, IB, K), jnp.int32),
        buf0=pltpu.VMEM((K, D), jnp.int16),
        buf1=pltpu.VMEM((K, D), jnp.int16),
        sbuf=pltpu.VMEM((SR // 5, D), jnp.int16),
        sidx=pltpu.VMEM((SR // 5,), jnp.int32),
        zbuf=pltpu.VMEM((ZR, D), jnp.int16),
        gsem0=pltpu.SemaphoreType.DMA,
        gsem1=pltpu.SemaphoreType.DMA,
        ssem0=pltpu.SemaphoreType.DMA,
        ssem1=pltpu.SemaphoreType.DMA,
        isem=pltpu.SemaphoreType.DMA,
    ),
)
def _agg_kernel(q_hbm, ei_hbm, out_hbm, *,
                acc, src_v, dst_v, buf0, buf1, sbuf, sidx, zbuf,
                gsem0, gsem1, ssem0, ssem1, isem):
    cid = lax.axis_index("c")
    sid = lax.axis_index("s")
    wid = sid * NC + cid
    bufs = (buf0, buf1)
    gsems = (gsem0, gsem1)
    ssems = (ssem0, ssem1)

    def zbody(i, _):
        for c in range(D // 32):
            zbuf[i, pl.ds(c * 32, 32)] = jnp.zeros((32,), jnp.int16)
        return 0

    lax.fori_loop(0, ZR, zbody, 0)
    for t in range(RPT // ZR):
        pltpu.sync_copy(zbuf, acc.at[pl.ds(sid * RPT + t * ZR, ZR)])
    plsc.subcore_barrier()

    SRC = SR // 5
    sidx_offs = list(range(0, SRC - 15, 16))
    if sidx_offs[-1] + 16 < SRC:
        sidx_offs.append(SRC - 16)

    @pl.when(cid == 0)
    def _():
        for t in range(5):
            base = sid * SR + t * SRC
            for off in sidx_offs:
                sidx[pl.ds(off, 16)] = lax.iota(jnp.int32, 16) + (base + off)
            pltpu.sync_copy(q_hbm.at[pl.ds(base, SRC)], sbuf)
            pltpu.sync_copy(sbuf, acc.at[sidx], add=True)

    def fetch_idx(b, slot):
        pltpu.async_copy(ei_hbm.at[0, wid, pl.ds(b * IB, IB)], src_v.at[slot], isem)
        pltpu.async_copy(ei_hbm.at[1, wid, pl.ds(b * IB, IB)], dst_v.at[slot], isem)

    def wait_idx(slot):
        pltpu.make_async_copy(ei_hbm.at[0, wid, pl.ds(0, IB)], src_v.at[slot], isem).wait()
        pltpu.make_async_copy(ei_hbm.at[1, wid, pl.ds(0, IB)], dst_v.at[slot], isem).wait()

    def start_gather(slot, r, p):
        pltpu.async_copy(q_hbm.at[src_v.at[slot, r]], bufs[p], gsems[p])

    def wait_gather(p):
        pltpu.make_async_copy(q_hbm.at[src_v.at[0, 0]], bufs[p], gsems[p]).wait()

    def start_scatter(slot, r, p):
        pltpu.async_copy(bufs[p], acc.at[dst_v.at[slot, r]], ssems[p], add=True)

    def wait_scatter(p):
        pltpu.make_async_copy(bufs[p], acc.at[dst_v.at[0, 0]], ssems[p]).wait()

    fetch_idx(0, 0)
    wait_idx(0)
    start_gather(0, 0, 0)

    def block(b, _):
        slot = lax.rem(b, 2)
        nslot = lax.rem(b + 1, 2)
        for t_ in range(IB):
            p = t_ % 2
            q = 1 - p
            wait_gather(p)
            start_scatter(slot, t_, p)
            if t_ == 0:
                @pl.when(b > 0)
                def _():
                    wait_scatter(q)

                @pl.when(b < NB - 1)
                def _():
                    fetch_idx(b + 1, nslot)
            else:
                wait_scatter(q)
            if t_ == IB - 1:
                @pl.when(b < NB - 1)
                def _():
                    wait_idx(nslot)
                    start_gather(nslot, 0, q)
            else:
                start_gather(slot, t_ + 1, q)
        return 0

    lax.fori_loop(0, NB, block, 0)
    wait_scatter((C - 1) % 2)
    plsc.subcore_barrier()

    pltpu.sync_copy(acc.at[pl.ds(sid * RPT, RPT)],
                    out_hbm.at[cid, pl.ds(sid * RPT, RPT)])


def _dinv_col(degp_ref):
    hist = degp_ref[...].sum(axis=0)
    deg = hist + 1.0
    dinv = lax.rsqrt(deg)
    return hist, dinv[:N, None]


def _scale_for(g, hist):
    cnt_max = jnp.max(hist) + 1.0
    maxg = jnp.max(jnp.abs(g))
    return (32767.0 - 0.5 * cnt_max - 2.0) / jnp.maximum(cnt_max * maxg, 1e-20)


def _tc1_body(degp_ref, x_ref, w_ref, g_ref, sc_ref):
    hist, dv = _dinv_col(degp_ref)
    h = jnp.dot(x_ref[...], w_ref[...], preferred_element_type=jnp.float32)
    g = h * dv
    g_ref[...] = g
    sc_ref[...] = jnp.full((1, 1), _scale_for(g, hist), jnp.float32)


def _tc2_body(accf_ref, degp_ref, b_ref, w_ref, g_ref, sc_ref):
    hist, dv = _dinv_col(degp_ref)
    out = dv * accf_ref[...] + b_ref[...]
    h = jnp.maximum(out, 0.0)
    g = dv * jnp.dot(h, w_ref[...], preferred_element_type=jnp.float32)
    g_ref[...] = g
    sc_ref[...] = jnp.full((1, 1), _scale_for(g, hist), jnp.float32)


def _tc3_body(accf_ref, degp_ref, b_ref, out_ref):
    _, dv = _dinv_col(degp_ref)
    out_ref[...] = dv * accf_ref[...] + b_ref[...]


def _quant_cast(g, sc):
    return jnp.clip(jnp.round(g * sc), -32767.0, 32767.0).astype(jnp.int16)


def _dequant_cast(accp, sc):
    p = jax.lax.bitcast_convert_type(
        accp.reshape(NC, NROWS, D // 2, 2), jnp.int32)
    lo = (p << 16) >> 16
    hi = p >> 16
    lo_s = lo[0, :N] + lo[1, :N]
    hi_s = hi[0, :N] + hi[1, :N]
    s = jnp.stack([lo_s, hi_s], axis=-1).reshape(N, D)
    return s.astype(jnp.float32) * (1.0 / sc)


def kernel(x, edge_index, W0, b0, W1, b1):
    ei4 = edge_index.reshape(2, NW, C, K)

    deg_p = _deg_kernel(ei4).reshape(NC, NROWS)

    g0, sc0 = pl.pallas_call(
        _tc1_body,
        out_shape=(
            jax.ShapeDtypeStruct((N, D), jnp.float32),
            jax.ShapeDtypeStruct((1, 1), jnp.float32),
        ),
    )(deg_p, x, W0)

    acc0 = _agg_kernel(_quant_cast(g0, sc0), ei4)

    g1, sc1 = pl.pallas_call(
        _tc2_body,
        out_shape=(
            jax.ShapeDtypeStruct((N, D), jnp.float32),
            jax.ShapeDtypeStruct((1, 1), jnp.float32),
        ),
    )(_dequant_cast(acc0, sc0), deg_p, b0.reshape(1, D), W1)

    acc1 = _agg_kernel(_quant_cast(g1, sc1), ei4)

    out = pl.pallas_call(
        _tc3_body,
        out_shape=jax.ShapeDtypeStruct((N, D), jnp.float32),
    )(_dequant_cast(acc1, sc1), deg_p, b1.reshape(1, D))

    return out
